# baseline (speedup 1.0000x reference)
import functools

import jax
import jax.numpy as jnp
from jax import lax
from jax.experimental import pallas as pl
from jax.experimental.pallas import tpu as pltpu
from jax.experimental.pallas import tpu_sc as plsc

N = 10000
E = 320000
IN = 16
H = 16
EA = 4

NC = 2
NS = 16
NW = NC * NS
PERW = E // NW
CH = 80
NCHUNK = PERW // CH
GRP = 5
NGRP = NCHUNK // GRP
ZROWS = N // NS

@functools.cache
def _mesh():
  return plsc.VectorSubcoreMesh(core_axis_name="c", subcore_axis_name="s")


_SC_PARAMS = pltpu.CompilerParams(use_tc_tiling_on_sc=False)


def _sc_gather(table, idx3, ea=None):
  outs = [jax.ShapeDtypeStruct((E, 16), jnp.float32)]
  scratch = [
      pltpu.VMEM((NCHUNK, CH), jnp.int32),
      pltpu.VMEM((GRP, CH, 16), jnp.float32),
      pltpu.SemaphoreType.DMA,
      pltpu.SemaphoreType.DMA,
  ]
  if ea is not None:
    outs.append(jax.ShapeDtypeStruct((E, EA), jnp.float32))
    scratch.append(pltpu.VMEM((PERW, EA), jnp.float32))

  @functools.partial(
      pl.kernel,
      mesh=_mesh(),
      out_type=outs,
      scratch_types=scratch,
      compiler_params=_SC_PARAMS,
  )
  def k(*refs):
    if ea is not None:
      (table_hbm, idx_hbm, ea_hbm, out_hbm, ea_out,
       idx_v, rows_v, gsem, ssem, ea_v) = refs
    else:
      table_hbm, idx_hbm, out_hbm, idx_v, rows_v, gsem, ssem = refs
    wid = lax.axis_index("s") * NC + lax.axis_index("c")
    base = wid * PERW
    pltpu.sync_copy(idx_hbm.at[wid], idx_v)
    if ea is not None:
      pltpu.sync_copy(ea_hbm.at[pl.ds(base, PERW)], ea_v)
      pltpu.sync_copy(ea_v, ea_out.at[pl.ds(base, PERW)])

    def group(g, carry):
      j0 = g * GRP

      @pl.when(g > 0)
      def _drain_stores():
        for b in range(GRP):
          pltpu.make_async_copy(
              rows_v.at[b], out_hbm.at[pl.ds(base, CH)], ssem).wait()

      for b in range(GRP):
        pltpu.async_copy(table_hbm.at[idx_v.at[j0 + b]], rows_v.at[b], gsem)
      for b in range(GRP):
        pltpu.make_async_copy(
            table_hbm.at[idx_v.at[j0 + b]], rows_v.at[b], gsem).wait()
      for b in range(GRP):
        pltpu.async_copy(
            rows_v.at[b], out_hbm.at[pl.ds(base + (j0 + b) * CH, CH)], ssem)
      return carry

    lax.fori_loop(0, NGRP, group, 0)
    for b in range(GRP):
      pltpu.make_async_copy(
          rows_v.at[b], out_hbm.at[pl.ds(base, CH)], ssem).wait()

  if ea is not None:
    return k(table, idx3, ea)
  return k(table, idx3)[0]


def _sc_scatter(msg, idx3, with_cnt):
  outs = [jax.ShapeDtypeStruct((NC, N, 16), jnp.float32)]
  scratch = [
      pltpu.VMEM((NCHUNK, CH), jnp.int32),
      pltpu.VMEM((GRP, CH, 16), jnp.float32),
      pltpu.VMEM((ZROWS, 16), jnp.float32),
      pltpu.VMEM_SHARED((N, 16), jnp.float32),
      pltpu.SemaphoreType.DMA,
      pltpu.SemaphoreType.DMA,
      pltpu.SemaphoreType.DMA,
  ]
  if with_cnt:
    outs.append(jax.ShapeDtypeStruct((NC, N, 16), jnp.float32))
    scratch.insert(2, pltpu.VMEM((CH, 16), jnp.float32))
    scratch.insert(4, pltpu.VMEM_SHARED((N, 16), jnp.float32))

  @functools.partial(
      pl.kernel, mesh=_mesh(), out_type=outs, scratch_types=scratch,
      compiler_params=_SC_PARAMS)
  def k(*refs):
    if with_cnt:
      (msg_hbm, idx_hbm, out_sum, out_cnt,
       idx_v, rows_v, ones_v, stage_v, acc, cacc, lsem, asem, csem) = refs
    else:
      (msg_hbm, idx_hbm, out_sum,
       idx_v, rows_v, stage_v, acc, lsem, asem, csem) = refs
    sid = lax.axis_index("s")
    cid = lax.axis_index("c")
    wid = sid * NC + cid
    base = wid * PERW

    def zbody(r, carry):
      stage_v[r, :] = jnp.zeros((16,), jnp.float32)
      return carry

    lax.fori_loop(0, ZROWS, zbody, 0)
    pltpu.sync_copy(stage_v, acc.at[pl.ds(sid * ZROWS, ZROWS)])
    if with_cnt:
      pltpu.sync_copy(stage_v, cacc.at[pl.ds(sid * ZROWS, ZROWS)])

      def obody(r, carry):
        ones_v[r, :] = jnp.ones((16,), jnp.float32)
        return carry

      lax.fori_loop(0, CH, obody, 0)
    pltpu.sync_copy(idx_hbm.at[wid], idx_v)
    plsc.subcore_barrier()

    def group(g, carry):
      j0 = g * GRP

      @pl.when(g > 0)
      def _drain_adds():
        for b in range(GRP):
          pltpu.make_async_copy(
              rows_v.at[b], acc.at[idx_v.at[j0 + b]], asem).wait()
          if with_cnt:
            pltpu.make_async_copy(
                ones_v, cacc.at[idx_v.at[j0 + b]], csem).wait()

      for b in range(GRP):
        pltpu.async_copy(
            msg_hbm.at[pl.ds(base + (j0 + b) * CH, CH)], rows_v.at[b], lsem)
      for b in range(GRP):
        pltpu.make_async_copy(
            msg_hbm.at[pl.ds(base, CH)], rows_v.at[b], lsem).wait()
      for b in range(GRP):
        pltpu.async_copy(rows_v.at[b], acc.at[idx_v.at[j0 + b]], asem,
                         add=True)
        if with_cnt:
          pltpu.async_copy(ones_v, cacc.at[idx_v.at[j0 + b]], csem, add=True)
      return carry

    lax.fori_loop(0, NGRP, group, 0)
    for b in range(GRP):
      pltpu.make_async_copy(rows_v.at[b], acc.at[idx_v.at[b]], asem).wait()
      if with_cnt:
        pltpu.make_async_copy(ones_v, cacc.at[idx_v.at[b]], csem).wait()
    plsc.subcore_barrier()

    pltpu.sync_copy(acc.at[pl.ds(sid * ZROWS, ZROWS)], stage_v)
    pltpu.sync_copy(stage_v, out_sum.at[cid, pl.ds(sid * ZROWS, ZROWS)])
    if with_cnt:
      pltpu.sync_copy(cacc.at[pl.ds(sid * ZROWS, ZROWS)], stage_v)
      pltpu.sync_copy(stage_v, out_cnt.at[cid, pl.ds(sid * ZROWS, ZROWS)])

  res = k(msg, idx3)
  return res if with_cnt else res[0]


_TB = 6400


def _tc_edge_msgs(eap, xjp, Wa_bd, ba8, Wb_bd, bb8, R_bd, S_bd):
  TB8 = _TB // 8

  def body(eap_ref, xj_ref, wa, ba_r, wb, bb_r, r_r, s_r, out_ref):
    hp = jnp.maximum(
        jnp.dot(eap_ref[...], wa[...], preferred_element_type=jnp.float32)
        + ba_r[...], 0.0)
    wp = jnp.dot(hp.astype(jnp.bfloat16), wb[...],
                 preferred_element_type=jnp.float32) + bb_r[...]
    xep = jnp.dot(xj_ref[...].astype(jnp.bfloat16), r_r[...],
                  preferred_element_type=jnp.float32)
    out_ref[...] = jnp.dot(
        (wp * xep).astype(jnp.bfloat16), s_r[...],
        preferred_element_type=jnp.float32)

  zero = lambda i: (0, 0)
  return pl.pallas_call(
      body,
      grid=(E // _TB,),
      in_specs=[
          pl.BlockSpec((TB8, 8 * EA), lambda i: (i, 0)),
          pl.BlockSpec((TB8, 128), lambda i: (i, 0)),
          pl.BlockSpec((8 * EA, 8 * 32), zero),
          pl.BlockSpec((1, 8 * 32), zero),
          pl.BlockSpec((8 * 32, 8 * IN * H), zero),
          pl.BlockSpec((1, 8 * IN * H), zero),
          pl.BlockSpec((128, 8 * IN * H), zero),
          pl.BlockSpec((8 * IN * H, 128), zero),
      ],
      out_specs=pl.BlockSpec((TB8, 128), lambda i: (i, 0)),
      out_shape=jax.ShapeDtypeStruct((E // 8, 128), jnp.float32),
  )(eap, xjp, Wa_bd, ba8.reshape(1, 8 * 32),
    Wb_bd.astype(jnp.bfloat16), bb8.reshape(1, 8 * IN * H),
    R_bd.astype(jnp.bfloat16), S_bd.astype(jnp.bfloat16))


NP8 = N // 8


def _tc_finalize1(sums_p, cnts_p, x_p, root_bd, bias_t):
  def body(s_ref, c_ref, x_ref, r_ref, b_ref, h_ref, rinv_ref):
    cnt = c_ref[0] + c_ref[1]
    rinv = 1.0 / jnp.maximum(cnt, 1.0)
    mean = (s_ref[0] + s_ref[1]) * rinv
    h = mean + jnp.dot(
        x_ref[...], r_ref[...], preferred_element_type=jnp.float32) + b_ref[...]
    h_ref[...] = jnp.maximum(h, 0.0)
    rinv_ref[...] = rinv

  return pl.pallas_call(
      body,
      out_shape=[
          jax.ShapeDtypeStruct((NP8, 128), jnp.float32),
          jax.ShapeDtypeStruct((NP8, 128), jnp.float32),
      ],
  )(sums_p, cnts_p, x_p, root_bd, bias_t.reshape(1, 128))


def _tc_finalize2(sums_p, rinv_p, h1_p, root_bd, bias_t, Wl_bd, bl_t):
  def body(s_ref, rinv_ref, h1_ref, r_ref, b_ref, wl_ref, bl_ref, out_ref):
    mean = (s_ref[0] + s_ref[1]) * rinv_ref[...]
    h2 = mean + jnp.dot(
        h1_ref[...], r_ref[...], preferred_element_type=jnp.float32) + b_ref[...]
    h2 = jnp.maximum(h2, 0.0)
    out_ref[...] = jnp.dot(
        h2, wl_ref[...], preferred_element_type=jnp.float32) + bl_ref[...]

  return pl.pallas_call(
      body,
      out_shape=jax.ShapeDtypeStruct((NP8, 8), jnp.float32),
  )(sums_p, rinv_p, h1_p, root_bd, bias_t.reshape(1, 128), Wl_bd,
    bl_t.reshape(1, 8))


def kernel(x, edge_index, edge_attr, W1a, b1a, W1b, b1b, root1, bias1,
           W2a, b2a, W2b, b2b, root2, bias2, Wl, bl):
  src3 = edge_index[0].astype(jnp.int32).reshape(NW, NCHUNK, CH)
  dst3 = edge_index[1].astype(jnp.int32).reshape(NW, NCHUNK, CH)

  c = jnp.arange(IN * H, dtype=jnp.int32)
  R = (jnp.arange(IN, dtype=jnp.int32)[:, None] == (c // H)[None, :]
       ).astype(jnp.float32)
  S = ((c % H)[:, None] == jnp.arange(H, dtype=jnp.int32)[None, :]
       ).astype(jnp.float32)

  eye8 = jnp.eye(8, dtype=jnp.float32)
  W1a_bd = jnp.kron(eye8, W1a)
  W1b_bd = jnp.kron(eye8, W1b)
  W2a_bd = jnp.kron(eye8, W2a)
  W2b_bd = jnp.kron(eye8, W2b)
  R_bd = jnp.kron(eye8, R)
  S_bd = jnp.kron(eye8, S)
  b1a8 = jnp.tile(b1a, 8)
  b1b8 = jnp.tile(b1b, 8)
  b2a8 = jnp.tile(b2a, 8)
  b2b8 = jnp.tile(b2b, 8)
  root1_bd = jnp.kron(eye8, root1)
  root2_bd = jnp.kron(eye8, root2)
  Wl_bd = jnp.kron(eye8, Wl)
  bias1_t = jnp.tile(bias1, 8)
  bias2_t = jnp.tile(bias2, 8)
  bl_t = jnp.tile(bl, 8)

  xj, ea_lin = _sc_gather(x, src3, edge_attr)
  eap = ea_lin.reshape(E // 8, 8 * EA)
  msg1 = _tc_edge_msgs(eap, xj.reshape(E // 8, 128),
                       W1a_bd, b1a8, W1b_bd, b1b8, R_bd, S_bd)
  sums1, cnts = _sc_scatter(msg1.reshape(E, 16), dst3, True)
  h1p, rinvp = _tc_finalize1(sums1.reshape(NC, NP8, 128),
                             cnts.reshape(NC, NP8, 128),
                             x.reshape(NP8, 128), root1_bd, bias1_t)

  xj2 = _sc_gather(h1p.reshape(N, H), src3)
  msg2 = _tc_edge_msgs(eap, xj2.reshape(E // 8, 128),
                       W2a_bd, b2a8, W2b_bd, b2b8, R_bd, S_bd)
  sums2 = _sc_scatter(msg2.reshape(E, 16), dst3, False)
  out = _tc_finalize2(sums2.reshape(NC, NP8, 128), rinvp, h1p,
                      root2_bd, bias2_t, Wl_bd, bl_t)
  return out.reshape(N)


# speedup vs baseline: 1.5350x; 1.5350x over previous
import functools

import jax
import jax.numpy as jnp
from jax import lax
from jax.experimental import pallas as pl
from jax.experimental.pallas import tpu as pltpu
from jax.experimental.pallas import tpu_sc as plsc

N = 10000
E = 320000
IN = 16
H = 16
EA = 4

NC = 2
NS = 16
NW = NC * NS
PERW = E // NW
CH = 80
NCHUNK = PERW // CH
GRP = 5
NGRP = NCHUNK // GRP
ZROWS = N // NS

@functools.cache
def _mesh():
  return plsc.VectorSubcoreMesh(core_axis_name="c", subcore_axis_name="s")


_SC_PARAMS = pltpu.CompilerParams(use_tc_tiling_on_sc=False)


def _sc_gather(table, idx3, ea=None):
  outs = [jax.ShapeDtypeStruct((E, 16), jnp.float32)]
  scratch = [
      pltpu.VMEM((NCHUNK, CH), jnp.int32),
      pltpu.VMEM((GRP, CH, 16), jnp.float32),
      pltpu.SemaphoreType.DMA,
      pltpu.SemaphoreType.DMA,
  ]
  if ea is not None:
    outs.append(jax.ShapeDtypeStruct((E, EA), jnp.float32))
    scratch.append(pltpu.VMEM((PERW, EA), jnp.float32))

  @functools.partial(
      pl.kernel,
      mesh=_mesh(),
      out_type=outs,
      scratch_types=scratch,
      compiler_params=_SC_PARAMS,
  )
  def k(*refs):
    if ea is not None:
      (table_hbm, idx_hbm, ea_hbm, out_hbm, ea_out,
       idx_v, rows_v, gsem, ssem, ea_v) = refs
    else:
      table_hbm, idx_hbm, out_hbm, idx_v, rows_v, gsem, ssem = refs
    wid = lax.axis_index("s") * NC + lax.axis_index("c")
    base = wid * PERW
    pltpu.sync_copy(idx_hbm.at[wid], idx_v)
    if ea is not None:
      pltpu.sync_copy(ea_hbm.at[pl.ds(base, PERW)], ea_v)
      pltpu.sync_copy(ea_v, ea_out.at[pl.ds(base, PERW)])

    def group(g, carry):
      j0 = g * GRP

      @pl.when(g > 0)
      def _drain_stores():
        for b in range(GRP):
          pltpu.make_async_copy(
              rows_v.at[b], out_hbm.at[pl.ds(base, CH)], ssem).wait()

      for b in range(GRP):
        pltpu.async_copy(table_hbm.at[idx_v.at[j0 + b]], rows_v.at[b], gsem)
      for b in range(GRP):
        pltpu.make_async_copy(
            table_hbm.at[idx_v.at[j0 + b]], rows_v.at[b], gsem).wait()
      for b in range(GRP):
        pltpu.async_copy(
            rows_v.at[b], out_hbm.at[pl.ds(base + (j0 + b) * CH, CH)], ssem)
      return carry

    lax.fori_loop(0, NGRP, group, 0)
    for b in range(GRP):
      pltpu.make_async_copy(
          rows_v.at[b], out_hbm.at[pl.ds(base, CH)], ssem).wait()

  if ea is not None:
    return k(table, idx3, ea)
  return k(table, idx3)[0]


def _sc_scatter(msg, idx3, with_cnt):
  outs = [jax.ShapeDtypeStruct((NC, N, 16), jnp.float32)]
  scratch = [
      pltpu.VMEM((NCHUNK, CH), jnp.int32),
      pltpu.VMEM((GRP, CH, 16), jnp.float32),
      pltpu.VMEM((ZROWS, 16), jnp.float32),
      pltpu.VMEM_SHARED((N, 16), jnp.float32),
      pltpu.SemaphoreType.DMA,
      pltpu.SemaphoreType.DMA,
      pltpu.SemaphoreType.DMA,
  ]
  if with_cnt:
    outs.append(jax.ShapeDtypeStruct((NC, N, 16), jnp.float32))
    scratch.insert(2, pltpu.VMEM((CH, 16), jnp.float32))
    scratch.insert(4, pltpu.VMEM_SHARED((N, 16), jnp.float32))

  @functools.partial(
      pl.kernel, mesh=_mesh(), out_type=outs, scratch_types=scratch,
      compiler_params=_SC_PARAMS)
  def k(*refs):
    if with_cnt:
      (msg_hbm, idx_hbm, out_sum, out_cnt,
       idx_v, rows_v, ones_v, stage_v, acc, cacc, lsem, asem, csem) = refs
    else:
      (msg_hbm, idx_hbm, out_sum,
       idx_v, rows_v, stage_v, acc, lsem, asem, csem) = refs
    sid = lax.axis_index("s")
    cid = lax.axis_index("c")
    wid = sid * NC + cid
    base = wid * PERW

    def zbody(r, carry):
      stage_v[r, :] = jnp.zeros((16,), jnp.float32)
      return carry

    lax.fori_loop(0, ZROWS, zbody, 0)
    pltpu.sync_copy(stage_v, acc.at[pl.ds(sid * ZROWS, ZROWS)])
    if with_cnt:
      pltpu.sync_copy(stage_v, cacc.at[pl.ds(sid * ZROWS, ZROWS)])

      def obody(r, carry):
        ones_v[r, :] = jnp.ones((16,), jnp.float32)
        return carry

      lax.fori_loop(0, CH, obody, 0)
    pltpu.sync_copy(idx_hbm.at[wid], idx_v)
    plsc.subcore_barrier()

    def group(g, carry):
      j0 = g * GRP

      @pl.when(g > 0)
      def _drain_adds():
        for b in range(GRP):
          pltpu.make_async_copy(
              rows_v.at[b], acc.at[idx_v.at[j0 + b]], asem).wait()
          if with_cnt:
            pltpu.make_async_copy(
                ones_v, cacc.at[idx_v.at[j0 + b]], csem).wait()

      for b in range(GRP):
        pltpu.async_copy(
            msg_hbm.at[pl.ds(base + (j0 + b) * CH, CH)], rows_v.at[b], lsem)
      for b in range(GRP):
        pltpu.make_async_copy(
            msg_hbm.at[pl.ds(base, CH)], rows_v.at[b], lsem).wait()
      for b in range(GRP):
        pltpu.async_copy(rows_v.at[b], acc.at[idx_v.at[j0 + b]], asem,
                         add=True)
        if with_cnt:
          pltpu.async_copy(ones_v, cacc.at[idx_v.at[j0 + b]], csem, add=True)
      return carry

    lax.fori_loop(0, NGRP, group, 0)
    for b in range(GRP):
      pltpu.make_async_copy(rows_v.at[b], acc.at[idx_v.at[b]], asem).wait()
      if with_cnt:
        pltpu.make_async_copy(ones_v, cacc.at[idx_v.at[b]], csem).wait()
    plsc.subcore_barrier()

    pltpu.sync_copy(acc.at[pl.ds(sid * ZROWS, ZROWS)], stage_v)
    pltpu.sync_copy(stage_v, out_sum.at[cid, pl.ds(sid * ZROWS, ZROWS)])
    if with_cnt:
      pltpu.sync_copy(cacc.at[pl.ds(sid * ZROWS, ZROWS)], stage_v)
      pltpu.sync_copy(stage_v, out_cnt.at[cid, pl.ds(sid * ZROWS, ZROWS)])

  res = k(msg, idx3)
  return res if with_cnt else res[0]


_TB = 6400


def _tc_edge_msgs(eap, xjp, Wa_bd, ba8, Wb_bd, bb8, R_bd, S_bd):
  TB8 = _TB // 8

  def body(eap_ref, xj_ref, wa, ba_r, wb, bb_r, r_r, s_r, out_ref):
    hp = jnp.maximum(
        jnp.dot(eap_ref[...], wa[...], preferred_element_type=jnp.float32)
        + ba_r[...], 0.0)
    hp = hp.astype(jnp.bfloat16)
    wp = jnp.dot(hp, wb[...],
                 preferred_element_type=jnp.float32) + bb_r[...]
    xep = jnp.dot(xj_ref[...].astype(jnp.bfloat16), r_r[...],
                  preferred_element_type=jnp.float32)
    out_ref[...] = jnp.dot(
        (wp * xep).astype(jnp.bfloat16), s_r[...],
        preferred_element_type=jnp.float32)

  zero = lambda i: (0, 0)
  return pl.pallas_call(
      body,
      grid=(E // _TB,),
      in_specs=[
          pl.BlockSpec((TB8, 8 * EA), lambda i: (i, 0)),
          pl.BlockSpec((TB8, 128), lambda i: (i, 0)),
          pl.BlockSpec((8 * EA, 8 * 32), zero),
          pl.BlockSpec((1, 8 * 32), zero),
          pl.BlockSpec((8 * 32, 8 * IN * H), zero),
          pl.BlockSpec((1, 8 * IN * H), zero),
          pl.BlockSpec((128, 8 * IN * H), zero),
          pl.BlockSpec((8 * IN * H, 128), zero),
      ],
      out_specs=pl.BlockSpec((TB8, 128), lambda i: (i, 0)),
      out_shape=jax.ShapeDtypeStruct((E // 8, 128), jnp.float32),
  )(eap, xjp, Wa_bd.astype(jnp.bfloat16), ba8.reshape(1, 8 * 32),
    Wb_bd.astype(jnp.bfloat16), bb8.reshape(1, 8 * IN * H),
    R_bd.astype(jnp.bfloat16), S_bd.astype(jnp.bfloat16))


NP8 = N // 8


def _tc_finalize1(sums_p, cnts_p, x_p, root_bd, bias_t):
  def body(s_ref, c_ref, x_ref, r_ref, b_ref, h_ref, rinv_ref):
    cnt = c_ref[0] + c_ref[1]
    rinv = 1.0 / jnp.maximum(cnt, 1.0)
    mean = (s_ref[0] + s_ref[1]) * rinv
    h = mean + jnp.dot(
        x_ref[...], r_ref[...], preferred_element_type=jnp.float32) + b_ref[...]
    h_ref[...] = jnp.maximum(h, 0.0)
    rinv_ref[...] = rinv

  return pl.pallas_call(
      body,
      out_shape=[
          jax.ShapeDtypeStruct((NP8, 128), jnp.float32),
          jax.ShapeDtypeStruct((NP8, 128), jnp.float32),
      ],
  )(sums_p, cnts_p, x_p, root_bd, bias_t.reshape(1, 128))


def _tc_finalize2(sums_p, rinv_p, h1_p, root_bd, bias_t, Wl_bd, bl_t):
  def body(s_ref, rinv_ref, h1_ref, r_ref, b_ref, wl_ref, bl_ref, out_ref):
    mean = (s_ref[0] + s_ref[1]) * rinv_ref[...]
    h2 = mean + jnp.dot(
        h1_ref[...], r_ref[...], preferred_element_type=jnp.float32) + b_ref[...]
    h2 = jnp.maximum(h2, 0.0)
    out_ref[...] = jnp.dot(
        h2, wl_ref[...], preferred_element_type=jnp.float32) + bl_ref[...]

  return pl.pallas_call(
      body,
      out_shape=jax.ShapeDtypeStruct((NP8, 8), jnp.float32),
  )(sums_p, rinv_p, h1_p, root_bd, bias_t.reshape(1, 128), Wl_bd,
    bl_t.reshape(1, 8))


def kernel(x, edge_index, edge_attr, W1a, b1a, W1b, b1b, root1, bias1,
           W2a, b2a, W2b, b2b, root2, bias2, Wl, bl):
  src3 = edge_index[0].astype(jnp.int32).reshape(NW, NCHUNK, CH)
  dst3 = edge_index[1].astype(jnp.int32).reshape(NW, NCHUNK, CH)

  c = jnp.arange(IN * H, dtype=jnp.int32)
  R = (jnp.arange(IN, dtype=jnp.int32)[:, None] == (c // H)[None, :]
       ).astype(jnp.float32)
  S = ((c % H)[:, None] == jnp.arange(H, dtype=jnp.int32)[None, :]
       ).astype(jnp.float32)

  eye8 = jnp.eye(8, dtype=jnp.float32)
  W1a_bd = jnp.kron(eye8, W1a)
  W1b_bd = jnp.kron(eye8, W1b)
  W2a_bd = jnp.kron(eye8, W2a)
  W2b_bd = jnp.kron(eye8, W2b)
  R_bd = jnp.kron(eye8, R)
  S_bd = jnp.kron(eye8, S)
  b1a8 = jnp.tile(b1a, 8)
  b1b8 = jnp.tile(b1b, 8)
  b2a8 = jnp.tile(b2a, 8)
  b2b8 = jnp.tile(b2b, 8)
  root1_bd = jnp.kron(eye8, root1)
  root2_bd = jnp.kron(eye8, root2)
  Wl_bd = jnp.kron(eye8, Wl)
  bias1_t = jnp.tile(bias1, 8)
  bias2_t = jnp.tile(bias2, 8)
  bl_t = jnp.tile(bl, 8)
  eap = edge_attr.astype(jnp.bfloat16).reshape(E // 8, 8 * EA)

  xj = _sc_gather(x, src3)
  msg1 = _tc_edge_msgs(eap, xj.reshape(E // 8, 128),
                       W1a_bd, b1a8, W1b_bd, b1b8, R_bd, S_bd)
  sums1, cnts = _sc_scatter(msg1.reshape(E, 16), dst3, True)
  h1p, rinvp = _tc_finalize1(sums1.reshape(NC, NP8, 128),
                             cnts.reshape(NC, NP8, 128),
                             x.reshape(NP8, 128), root1_bd, bias1_t)

  xj2 = _sc_gather(h1p.reshape(N, H), src3)
  msg2 = _tc_edge_msgs(eap, xj2.reshape(E // 8, 128),
                       W2a_bd, b2a8, W2b_bd, b2b8, R_bd, S_bd)
  sums2 = _sc_scatter(msg2.reshape(E, 16), dst3, False)
  out = _tc_finalize2(sums2.reshape(NC, NP8, 128), rinvp, h1p,
                      root2_bd, bias2_t, Wl_bd, bl_t)
  return out.reshape(N)
